# Initial kernel scaffold; baseline (speedup 1.0000x reference)
import jax
import jax.numpy as jnp
from jax.experimental import pallas as pl


def kernel(x, edge_index, input_W, input_b, msg_W, msg_b, gru_wih, gru_whh, gru_bih, gru_bhh, mu_W, mu_b, ls_W, ls_b):
    raise NotImplementedError("write your pallas kernel here")


# speedup vs baseline: 9.5465x; 9.5465x over previous
import functools

import jax
import jax.numpy as jnp
from jax import lax
from jax.experimental import pallas as pl
from jax.experimental.pallas import tpu as pltpu
from jax.experimental.pallas import tpu_sc as plsc

N_NODES = 100000
N_EDGES = 1600000
FDIM = 128
SDIM = 32
HDIM = 16
GDIM = 96
LDIM = 16
ROUNDS = 3

NC = 2
NS = 16
CHUNK = 128
BLK = 8

CHUNKS_TOTAL = -(-N_EDGES // (CHUNK * BLK * NS)) * BLK * NS
CHUNKS_PER_TILE = CHUNKS_TOTAL // NS
BLOCKS_PER_TILE = CHUNKS_PER_TILE // BLK
E_PAD = CHUNKS_TOTAL * CHUNK

ACC_ROWS = 100096
ZERO_PER_TILE = ACC_ROWS // NS
TRASH_ROW = N_NODES

ROW_T = 2000
GRID = N_NODES // ROW_T


def _dot(a, b):
    return jnp.dot(a, b, preferred_element_type=jnp.float32)


def _tc_init_body(x_ref, inW_ref, inb_ref, mW_ref, mb_ref, whhT_ref, bhh_ref,
                  state_ref, mlo_ref, mhi_ref, gh_ref):
    st = jnp.maximum(_dot(x_ref[...], inW_ref[...]) + inb_ref[...], 0.0)
    state_ref[...] = st
    m = jnp.maximum(_dot(st, mW_ref[...]) + mb_ref[...], 0.0)
    mlo_ref[...] = m[:, :HDIM]
    mhi_ref[...] = m[:, HDIM:]
    gh_ref[...] = _dot(st, whhT_ref[...]) + bhh_ref[...]


def _gru_new_state(state_ref, alo_ref, ahi_ref, gh_ref, wihT_ref, bih_ref):
    w = wihT_ref[...]
    gi = (_dot(alo_ref[...], w[:HDIM, :]) + _dot(ahi_ref[...], w[HDIM:, :])
          + bih_ref[...])
    h = state_ref[...]
    gh = gh_ref[...]
    r = jax.nn.sigmoid(gi[:, :SDIM] + gh[:, :SDIM])
    z = jax.nn.sigmoid(gi[:, SDIM:2 * SDIM] + gh[:, SDIM:2 * SDIM])
    n = jnp.tanh(gi[:, 2 * SDIM:] + r * gh[:, 2 * SDIM:])
    return h + (1.0 - z) * n + z * h


def _tc_mid_body(state_ref, alo_ref, ahi_ref, gh_ref, wihT_ref, bih_ref,
                 mW_ref, mb_ref, whhT_ref, bhh_ref,
                 nstate_ref, mlo_ref, mhi_ref, ghn_ref):
    new = _gru_new_state(state_ref, alo_ref, ahi_ref, gh_ref, wihT_ref, bih_ref)
    nstate_ref[...] = new
    m = jnp.maximum(_dot(new, mW_ref[...]) + mb_ref[...], 0.0)
    mlo_ref[...] = m[:, :HDIM]
    mhi_ref[...] = m[:, HDIM:]
    ghn_ref[...] = _dot(new, whhT_ref[...]) + bhh_ref[...]


def _tc_final_body(state_ref, alo_ref, ahi_ref, gh_ref, wihT_ref, bih_ref,
                   muW_ref, mub_ref, lsW_ref, lsb_ref, mu_ref, ls_ref):
    new = _gru_new_state(state_ref, alo_ref, ahi_ref, gh_ref, wihT_ref, bih_ref)
    mu_ref[...] = _dot(new, muW_ref[...]) + mub_ref[...]
    ls_ref[...] = _dot(new, lsW_ref[...]) + lsb_ref[...]


def _row_spec(width):
    return pl.BlockSpec((ROW_T, width), lambda i: (i, 0))


def _full_spec(shape):
    return pl.BlockSpec(shape, lambda i: (0,) * len(shape))


def _sds(*shape):
    return jax.ShapeDtypeStruct(shape, jnp.float32)


def _tc_init(x, inW, inb, mW, mb, whhT, bhh):
    return pl.pallas_call(
        _tc_init_body,
        grid=(GRID,),
        in_specs=[_row_spec(FDIM), _full_spec((FDIM, SDIM)), _full_spec((1, SDIM)),
                  _full_spec((SDIM, SDIM)), _full_spec((1, SDIM)),
                  _full_spec((SDIM, GDIM)), _full_spec((1, GDIM))],
        out_specs=[_row_spec(SDIM), _row_spec(HDIM), _row_spec(HDIM),
                   _row_spec(GDIM)],
        out_shape=[_sds(N_NODES, SDIM), _sds(N_NODES, HDIM),
                   _sds(N_NODES, HDIM), _sds(N_NODES, GDIM)],
    )(x, inW, inb, mW, mb, whhT, bhh)


def _tc_mid(state, alo, ahi, gh, wihT, bih, mW, mb, whhT, bhh):
    return pl.pallas_call(
        _tc_mid_body,
        grid=(GRID,),
        in_specs=[_row_spec(SDIM), _row_spec(HDIM), _row_spec(HDIM),
                  _row_spec(GDIM), _full_spec((SDIM, GDIM)), _full_spec((1, GDIM)),
                  _full_spec((SDIM, SDIM)), _full_spec((1, SDIM)),
                  _full_spec((SDIM, GDIM)), _full_spec((1, GDIM))],
        out_specs=[_row_spec(SDIM), _row_spec(HDIM), _row_spec(HDIM),
                   _row_spec(GDIM)],
        out_shape=[_sds(N_NODES, SDIM), _sds(N_NODES, HDIM),
                   _sds(N_NODES, HDIM), _sds(N_NODES, GDIM)],
    )(state, alo, ahi, gh, wihT, bih, mW, mb, whhT, bhh)


def _tc_final(state, alo, ahi, gh, wihT, bih, muW, mub, lsW, lsb):
    return pl.pallas_call(
        _tc_final_body,
        grid=(GRID,),
        in_specs=[_row_spec(SDIM), _row_spec(HDIM), _row_spec(HDIM),
                  _row_spec(GDIM), _full_spec((SDIM, GDIM)), _full_spec((1, GDIM)),
                  _full_spec((SDIM, LDIM)), _full_spec((1, LDIM)),
                  _full_spec((SDIM, LDIM)), _full_spec((1, LDIM))],
        out_specs=[_row_spec(LDIM), _row_spec(LDIM)],
        out_shape=[_sds(N_NODES, LDIM), _sds(N_NODES, LDIM)],
    )(state, alo, ahi, gh, wihT, bih, muW, mub, lsW, lsb)


def _sc_body(mlo_hbm, mhi_hbm, src_hbm, dst_hbm, alo_hbm, ahi_hbm,
             acc, sv, dv, rows, zbuf, obuf, gsem, ssem):
    c = lax.axis_index("c")
    s = lax.axis_index("s")

    def _zrow(i, _):
        zbuf[i, pl.ds(0, HDIM)] = jnp.zeros((HDIM,), jnp.float32)
        return 0
    lax.fori_loop(0, CHUNK, _zrow, 0)
    zbase = s * ZERO_PER_TILE
    nfull = ZERO_PER_TILE // CHUNK
    rem = ZERO_PER_TILE - nfull * CHUNK

    def _zchunk(k, _):
        pltpu.sync_copy(zbuf, acc.at[pl.ds(zbase + k * CHUNK, CHUNK)])
        return 0
    lax.fori_loop(0, nfull, _zchunk, 0)
    if rem:
        pltpu.sync_copy(zbuf.at[pl.ds(0, rem)],
                        acc.at[pl.ds(zbase + nfull * CHUNK, rem)])
    plsc.subcore_barrier()

    def _accumulate(msg_ref):
        cbase = s * CHUNKS_PER_TILE

        def _block(b, _):
            blk0 = cbase + b * BLK
            pltpu.sync_copy(src_hbm.at[pl.ds(blk0, BLK)], sv)
            pltpu.sync_copy(dst_hbm.at[pl.ds(blk0, BLK)], dv)
            gathers = [pltpu.async_copy(msg_ref.at[sv.at[j]], rows.at[j], gsem)
                       for j in range(BLK)]
            for cp in gathers:
                cp.wait()
            scatters = [pltpu.async_copy(rows.at[j], acc.at[dv.at[j]], ssem,
                                         add=True)
                        for j in range(BLK)]
            for cp in scatters:
                cp.wait()
            return 0
        lax.fori_loop(0, BLOCKS_PER_TILE, _block, 0)

    @pl.when(c == 0)
    def _():
        _accumulate(mlo_hbm)

    @pl.when(c == 1)
    def _():
        _accumulate(mhi_hbm)

    plsc.subcore_barrier()

    def _writeback(out_ref):
        wbase = s * ZERO_PER_TILE
        wfull = ZERO_PER_TILE // CHUNK
        wrem = ZERO_PER_TILE - wfull * CHUNK

        def _wchunk(k, _):
            off = wbase + k * CHUNK
            pltpu.sync_copy(acc.at[pl.ds(off, CHUNK)], obuf)
            pltpu.sync_copy(obuf, out_ref.at[pl.ds(off, CHUNK)])
            return 0
        lax.fori_loop(0, wfull, _wchunk, 0)
        if wrem:
            off = wbase + wfull * CHUNK
            pltpu.sync_copy(acc.at[pl.ds(off, wrem)], obuf.at[pl.ds(0, wrem)])
            pltpu.sync_copy(obuf.at[pl.ds(0, wrem)], out_ref.at[pl.ds(off, wrem)])

    @pl.when(c == 0)
    def _():
        _writeback(alo_hbm)

    @pl.when(c == 1)
    def _():
        _writeback(ahi_hbm)


@functools.cache
def _sc_aggregate_fn():
    return pl.kernel(
        _sc_body,
        out_type=[_sds(ACC_ROWS, HDIM), _sds(ACC_ROWS, HDIM)],
        mesh=plsc.VectorSubcoreMesh(core_axis_name="c", subcore_axis_name="s"),
        scratch_types=[
            pltpu.VMEM_SHARED((ACC_ROWS, HDIM), jnp.float32),
            pltpu.VMEM((BLK, CHUNK), jnp.int32),
            pltpu.VMEM((BLK, CHUNK), jnp.int32),
            pltpu.VMEM((BLK, CHUNK, HDIM), jnp.float32),
            pltpu.VMEM((CHUNK, HDIM), jnp.float32),
            pltpu.VMEM((CHUNK, HDIM), jnp.float32),
            pltpu.SemaphoreType.DMA,
            pltpu.SemaphoreType.DMA,
        ],
        compiler_params=pltpu.CompilerParams(use_tc_tiling_on_sc=False),
    )


def _sc_aggregate(mlo, mhi, src2, dst2):
    alo, ahi = _sc_aggregate_fn()(mlo, mhi, src2, dst2)
    return alo[:N_NODES], ahi[:N_NODES]


def kernel(x, edge_index, input_W, input_b, msg_W, msg_b, gru_wih, gru_whh,
           gru_bih, gru_bhh, mu_W, mu_b, ls_W, ls_b):
    pad = E_PAD - N_EDGES
    src = jnp.concatenate([edge_index[0], jnp.zeros((pad,), jnp.int32)])
    dst = jnp.concatenate([edge_index[1],
                           jnp.full((pad,), TRASH_ROW, jnp.int32)])
    src2 = src.reshape(CHUNKS_TOTAL, CHUNK)
    dst2 = dst.reshape(CHUNKS_TOTAL, CHUNK)

    inb = input_b.reshape(1, SDIM)
    mb = msg_b.reshape(ROUNDS, 1, SDIM)
    bih = gru_bih.reshape(ROUNDS, 1, GDIM)
    bhh = gru_bhh.reshape(ROUNDS, 1, GDIM)
    wihT = jnp.transpose(gru_wih, (0, 2, 1))
    whhT = jnp.transpose(gru_whh, (0, 2, 1))
    mub = mu_b.reshape(1, LDIM)
    lsb = ls_b.reshape(1, LDIM)

    state, mlo, mhi, gh = _tc_init(x, input_W, inb, msg_W[0], mb[0],
                                   whhT[0], bhh[0])
    for r in range(ROUNDS):
        alo, ahi = _sc_aggregate(mlo, mhi, src2, dst2)
        if r < ROUNDS - 1:
            state, mlo, mhi, gh = _tc_mid(state, alo, ahi, gh, wihT[r], bih[r],
                                          msg_W[r + 1], mb[r + 1],
                                          whhT[r + 1], bhh[r + 1])
        else:
            mu, ls = _tc_final(state, alo, ahi, gh, wihT[r], bih[r],
                               mu_W, mub, ls_W, lsb)
    return (mu, ls)


# speedup vs baseline: 9.9693x; 1.0443x over previous
import functools

import jax
import jax.numpy as jnp
from jax import lax
from jax.experimental import pallas as pl
from jax.experimental.pallas import tpu as pltpu
from jax.experimental.pallas import tpu_sc as plsc

N_NODES = 100000
N_EDGES = 1600000
FDIM = 128
SDIM = 32
HDIM = 16
GDIM = 96
LDIM = 16
ROUNDS = 3

NC = 2
NS = 16
CHUNK = 128
BLK = 6

CHUNKS_PER_TILE = -(-N_EDGES // (CHUNK * 2 * BLK * NS)) * 2 * BLK
CHUNKS_TOTAL = CHUNKS_PER_TILE * NS
BLOCKS_PER_TILE = CHUNKS_PER_TILE // BLK
E_PAD = CHUNKS_TOTAL * CHUNK

ACC_ROWS = 100096
ZERO_PER_TILE = ACC_ROWS // NS
TRASH_ROW = N_NODES

ROW_T = 2000
GRID = N_NODES // ROW_T


def _dot(a, b):
    return jnp.dot(a, b, preferred_element_type=jnp.float32)


def _tc_init_body(x_ref, inW_ref, inb_ref, mW_ref, mb_ref, whhT_ref, bhh_ref,
                  state_ref, mlo_ref, mhi_ref, gh_ref):
    st = jnp.maximum(_dot(x_ref[...], inW_ref[...]) + inb_ref[...], 0.0)
    state_ref[...] = st
    m = jnp.maximum(_dot(st, mW_ref[...]) + mb_ref[...], 0.0)
    mlo_ref[...] = m[:, :HDIM]
    mhi_ref[...] = m[:, HDIM:]
    gh_ref[...] = _dot(st, whhT_ref[...]) + bhh_ref[...]


def _gru_new_state(state_ref, alo_ref, ahi_ref, gh_ref, wihT_ref, bih_ref):
    w = wihT_ref[...]
    gi = (_dot(alo_ref[...], w[:HDIM, :]) + _dot(ahi_ref[...], w[HDIM:, :])
          + bih_ref[...])
    h = state_ref[...]
    gh = gh_ref[...]
    r = jax.nn.sigmoid(gi[:, :SDIM] + gh[:, :SDIM])
    z = jax.nn.sigmoid(gi[:, SDIM:2 * SDIM] + gh[:, SDIM:2 * SDIM])
    n = jnp.tanh(gi[:, 2 * SDIM:] + r * gh[:, 2 * SDIM:])
    return h + (1.0 - z) * n + z * h


def _tc_mid_body(state_ref, alo_ref, ahi_ref, gh_ref, wihT_ref, bih_ref,
                 mW_ref, mb_ref, whhT_ref, bhh_ref,
                 nstate_ref, mlo_ref, mhi_ref, ghn_ref):
    new = _gru_new_state(state_ref, alo_ref, ahi_ref, gh_ref, wihT_ref, bih_ref)
    nstate_ref[...] = new
    m = jnp.maximum(_dot(new, mW_ref[...]) + mb_ref[...], 0.0)
    mlo_ref[...] = m[:, :HDIM]
    mhi_ref[...] = m[:, HDIM:]
    ghn_ref[...] = _dot(new, whhT_ref[...]) + bhh_ref[...]


def _tc_final_body(state_ref, alo_ref, ahi_ref, gh_ref, wihT_ref, bih_ref,
                   muW_ref, mub_ref, lsW_ref, lsb_ref, mu_ref, ls_ref):
    new = _gru_new_state(state_ref, alo_ref, ahi_ref, gh_ref, wihT_ref, bih_ref)
    mu_ref[...] = _dot(new, muW_ref[...]) + mub_ref[...]
    ls_ref[...] = _dot(new, lsW_ref[...]) + lsb_ref[...]


def _row_spec(width):
    return pl.BlockSpec((ROW_T, width), lambda i: (i, 0))


def _full_spec(shape):
    return pl.BlockSpec(shape, lambda i: (0,) * len(shape))


def _sds(*shape):
    return jax.ShapeDtypeStruct(shape, jnp.float32)


def _tc_init(x, inW, inb, mW, mb, whhT, bhh):
    return pl.pallas_call(
        _tc_init_body,
        grid=(GRID,),
        in_specs=[_row_spec(FDIM), _full_spec((FDIM, SDIM)), _full_spec((1, SDIM)),
                  _full_spec((SDIM, SDIM)), _full_spec((1, SDIM)),
                  _full_spec((SDIM, GDIM)), _full_spec((1, GDIM))],
        out_specs=[_row_spec(SDIM), _row_spec(HDIM), _row_spec(HDIM),
                   _row_spec(GDIM)],
        out_shape=[_sds(N_NODES, SDIM), _sds(N_NODES, HDIM),
                   _sds(N_NODES, HDIM), _sds(N_NODES, GDIM)],
    )(x, inW, inb, mW, mb, whhT, bhh)


def _tc_mid(state, alo, ahi, gh, wihT, bih, mW, mb, whhT, bhh):
    return pl.pallas_call(
        _tc_mid_body,
        grid=(GRID,),
        in_specs=[_row_spec(SDIM), _row_spec(HDIM), _row_spec(HDIM),
                  _row_spec(GDIM), _full_spec((SDIM, GDIM)), _full_spec((1, GDIM)),
                  _full_spec((SDIM, SDIM)), _full_spec((1, SDIM)),
                  _full_spec((SDIM, GDIM)), _full_spec((1, GDIM))],
        out_specs=[_row_spec(SDIM), _row_spec(HDIM), _row_spec(HDIM),
                   _row_spec(GDIM)],
        out_shape=[_sds(N_NODES, SDIM), _sds(N_NODES, HDIM),
                   _sds(N_NODES, HDIM), _sds(N_NODES, GDIM)],
    )(state, alo, ahi, gh, wihT, bih, mW, mb, whhT, bhh)


def _tc_final(state, alo, ahi, gh, wihT, bih, muW, mub, lsW, lsb):
    return pl.pallas_call(
        _tc_final_body,
        grid=(GRID,),
        in_specs=[_row_spec(SDIM), _row_spec(HDIM), _row_spec(HDIM),
                  _row_spec(GDIM), _full_spec((SDIM, GDIM)), _full_spec((1, GDIM)),
                  _full_spec((SDIM, LDIM)), _full_spec((1, LDIM)),
                  _full_spec((SDIM, LDIM)), _full_spec((1, LDIM))],
        out_specs=[_row_spec(LDIM), _row_spec(LDIM)],
        out_shape=[_sds(N_NODES, LDIM), _sds(N_NODES, LDIM)],
    )(state, alo, ahi, gh, wihT, bih, muW, mub, lsW, lsb)


def _sc_body(mlo_hbm, mhi_hbm, idx_hbm, zero_hbm, alo_hbm, ahi_hbm,
             acc, iv0, iv1, rows0, rows1, gsem, ssem):
    c = lax.axis_index("c")
    s = lax.axis_index("s")
    stripe = s * ZERO_PER_TILE

    pltpu.sync_copy(zero_hbm.at[pl.ds(stripe, ZERO_PER_TILE)],
                    acc.at[pl.ds(stripe, ZERO_PER_TILE)])
    plsc.subcore_barrier()

    def _accumulate(msg_ref):
        cbase = s * CHUNKS_PER_TILE

        def _fire_block(iv, rows, blk0):
            pltpu.sync_copy(idx_hbm.at[pl.ds(blk0, BLK)], iv)
            for j in range(BLK):
                pltpu.async_copy(msg_ref.at[iv.at[j, 0]], rows.at[j], gsem)

        def _drain_g_fire_s(iv, rows):
            for j in range(BLK):
                pltpu.make_async_copy(msg_ref.at[iv.at[j, 0]], rows.at[j],
                                      gsem).wait()
                pltpu.async_copy(rows.at[j], acc.at[iv.at[j, 1]], ssem,
                                 add=True)

        def _drain_s(iv, rows):
            for j in range(BLK):
                pltpu.make_async_copy(rows.at[j], acc.at[iv.at[j, 1]],
                                      ssem).wait()

        _fire_block(iv0, rows0, cbase)
        _fire_block(iv1, rows1, cbase + BLK)

        def _pair(p, _):
            b0 = cbase + (2 * p) * BLK
            _drain_g_fire_s(iv0, rows0)
            _drain_g_fire_s(iv1, rows1)
            _drain_s(iv0, rows0)
            _fire_block(iv0, rows0, b0 + 2 * BLK)
            _drain_s(iv1, rows1)
            _fire_block(iv1, rows1, b0 + 3 * BLK)
            return 0
        lax.fori_loop(0, BLOCKS_PER_TILE // 2 - 1, _pair, 0)
        _drain_g_fire_s(iv0, rows0)
        _drain_g_fire_s(iv1, rows1)
        _drain_s(iv0, rows0)
        _drain_s(iv1, rows1)

    @pl.when(c == 0)
    def _():
        _accumulate(mlo_hbm)

    @pl.when(c == 1)
    def _():
        _accumulate(mhi_hbm)

    plsc.subcore_barrier()

    @pl.when(c == 0)
    def _():
        pltpu.sync_copy(acc.at[pl.ds(stripe, ZERO_PER_TILE)],
                        alo_hbm.at[pl.ds(stripe, ZERO_PER_TILE)])

    @pl.when(c == 1)
    def _():
        pltpu.sync_copy(acc.at[pl.ds(stripe, ZERO_PER_TILE)],
                        ahi_hbm.at[pl.ds(stripe, ZERO_PER_TILE)])


@functools.cache
def _sc_aggregate_fn():
    return pl.kernel(
        _sc_body,
        out_type=[_sds(ACC_ROWS, HDIM), _sds(ACC_ROWS, HDIM)],
        mesh=plsc.VectorSubcoreMesh(core_axis_name="c", subcore_axis_name="s"),
        scratch_types=[
            pltpu.VMEM_SHARED((ACC_ROWS, HDIM), jnp.float32),
            pltpu.VMEM((BLK, 2, CHUNK), jnp.int32),
            pltpu.VMEM((BLK, 2, CHUNK), jnp.int32),
            pltpu.VMEM((BLK, CHUNK, HDIM), jnp.float32),
            pltpu.VMEM((BLK, CHUNK, HDIM), jnp.float32),
            pltpu.SemaphoreType.DMA,
            pltpu.SemaphoreType.DMA,
        ],
        compiler_params=pltpu.CompilerParams(use_tc_tiling_on_sc=False),
    )


def _sc_aggregate(mlo, mhi, idx_comb, zeros):
    alo, ahi = _sc_aggregate_fn()(mlo, mhi, idx_comb, zeros)
    return alo[:N_NODES], ahi[:N_NODES]


def kernel(x, edge_index, input_W, input_b, msg_W, msg_b, gru_wih, gru_whh,
           gru_bih, gru_bhh, mu_W, mu_b, ls_W, ls_b):
    pad = E_PAD - N_EDGES
    src = jnp.concatenate([edge_index[0], jnp.zeros((pad,), jnp.int32)])
    dst = jnp.concatenate([edge_index[1],
                           jnp.full((pad,), TRASH_ROW, jnp.int32)])
    idx_comb = jnp.stack([src.reshape(CHUNKS_TOTAL, CHUNK),
                          dst.reshape(CHUNKS_TOTAL, CHUNK)], axis=1)
    zeros = jnp.zeros((ACC_ROWS, HDIM), jnp.float32)

    inb = input_b.reshape(1, SDIM)
    mb = msg_b.reshape(ROUNDS, 1, SDIM)
    bih = gru_bih.reshape(ROUNDS, 1, GDIM)
    bhh = gru_bhh.reshape(ROUNDS, 1, GDIM)
    wihT = jnp.transpose(gru_wih, (0, 2, 1))
    whhT = jnp.transpose(gru_whh, (0, 2, 1))
    mub = mu_b.reshape(1, LDIM)
    lsb = ls_b.reshape(1, LDIM)

    state, mlo, mhi, gh = _tc_init(x, input_W, inb, msg_W[0], mb[0],
                                   whhT[0], bhh[0])
    for r in range(ROUNDS):
        alo, ahi = _sc_aggregate(mlo, mhi, idx_comb, zeros)
        if r < ROUNDS - 1:
            state, mlo, mhi, gh = _tc_mid(state, alo, ahi, gh, wihT[r], bih[r],
                                          msg_W[r + 1], mb[r + 1],
                                          whhT[r + 1], bhh[r + 1])
        else:
            mu, ls = _tc_final(state, alo, ahi, gh, wihT[r], bih[r],
                               mu_W, mub, ls_W, lsb)
    return (mu, ls)


# speedup vs baseline: 10.4265x; 1.0459x over previous
import functools

import jax
import jax.numpy as jnp
from jax import lax
from jax.experimental import pallas as pl
from jax.experimental.pallas import tpu as pltpu
from jax.experimental.pallas import tpu_sc as plsc

N_NODES = 100000
N_EDGES = 1600000
FDIM = 128
SDIM = 32
HDIM = 16
GDIM = 96
LDIM = 16
ROUNDS = 3

NC = 2
NS = 16
CHUNK = 256
BLK = 3

CHUNKS_PER_TILE = -(-N_EDGES // (CHUNK * 2 * BLK * NS)) * 2 * BLK
CHUNKS_TOTAL = CHUNKS_PER_TILE * NS
BLOCKS_PER_TILE = CHUNKS_PER_TILE // BLK
E_PAD = CHUNKS_TOTAL * CHUNK

ACC_ROWS = 100096
ZERO_PER_TILE = ACC_ROWS // NS
TRASH_ROW = N_NODES

ROW_T = 2000
GRID = N_NODES // ROW_T


def _dot(a, b):
    return jnp.dot(a, b, preferred_element_type=jnp.float32)


def _tc_init_body(x_ref, inW_ref, inb_ref, mW_ref, mb_ref, whhT_ref, bhh_ref,
                  state_ref, mlo_ref, mhi_ref, gh_ref):
    st = jnp.maximum(_dot(x_ref[...], inW_ref[...]) + inb_ref[...], 0.0)
    state_ref[...] = st
    m = jnp.maximum(_dot(st, mW_ref[...]) + mb_ref[...], 0.0)
    mlo_ref[...] = m[:, :HDIM]
    mhi_ref[...] = m[:, HDIM:]
    gh_ref[...] = _dot(st, whhT_ref[...]) + bhh_ref[...]


def _gru_new_state(state_ref, alo_ref, ahi_ref, gh_ref, wihT_ref, bih_ref):
    w = wihT_ref[...]
    gi = (_dot(alo_ref[...], w[:HDIM, :]) + _dot(ahi_ref[...], w[HDIM:, :])
          + bih_ref[...])
    h = state_ref[...]
    gh = gh_ref[...]
    r = jax.nn.sigmoid(gi[:, :SDIM] + gh[:, :SDIM])
    z = jax.nn.sigmoid(gi[:, SDIM:2 * SDIM] + gh[:, SDIM:2 * SDIM])
    n = jnp.tanh(gi[:, 2 * SDIM:] + r * gh[:, 2 * SDIM:])
    return h + (1.0 - z) * n + z * h


def _tc_mid_body(state_ref, alo_ref, ahi_ref, gh_ref, wihT_ref, bih_ref,
                 mW_ref, mb_ref, whhT_ref, bhh_ref,
                 nstate_ref, mlo_ref, mhi_ref, ghn_ref):
    new = _gru_new_state(state_ref, alo_ref, ahi_ref, gh_ref, wihT_ref, bih_ref)
    nstate_ref[...] = new
    m = jnp.maximum(_dot(new, mW_ref[...]) + mb_ref[...], 0.0)
    mlo_ref[...] = m[:, :HDIM]
    mhi_ref[...] = m[:, HDIM:]
    ghn_ref[...] = _dot(new, whhT_ref[...]) + bhh_ref[...]


def _tc_final_body(state_ref, alo_ref, ahi_ref, gh_ref, wihT_ref, bih_ref,
                   muW_ref, mub_ref, lsW_ref, lsb_ref, mu_ref, ls_ref):
    new = _gru_new_state(state_ref, alo_ref, ahi_ref, gh_ref, wihT_ref, bih_ref)
    mu_ref[...] = _dot(new, muW_ref[...]) + mub_ref[...]
    ls_ref[...] = _dot(new, lsW_ref[...]) + lsb_ref[...]


def _row_spec(width):
    return pl.BlockSpec((ROW_T, width), lambda i: (i, 0))


def _full_spec(shape):
    return pl.BlockSpec(shape, lambda i: (0,) * len(shape))


def _sds(*shape):
    return jax.ShapeDtypeStruct(shape, jnp.float32)


def _tc_init(x, inW, inb, mW, mb, whhT, bhh):
    return pl.pallas_call(
        _tc_init_body,
        grid=(GRID,),
        in_specs=[_row_spec(FDIM), _full_spec((FDIM, SDIM)), _full_spec((1, SDIM)),
                  _full_spec((SDIM, SDIM)), _full_spec((1, SDIM)),
                  _full_spec((SDIM, GDIM)), _full_spec((1, GDIM))],
        out_specs=[_row_spec(SDIM), _row_spec(HDIM), _row_spec(HDIM),
                   _row_spec(GDIM)],
        out_shape=[_sds(N_NODES, SDIM), _sds(N_NODES, HDIM),
                   _sds(N_NODES, HDIM), _sds(N_NODES, GDIM)],
    )(x, inW, inb, mW, mb, whhT, bhh)


def _tc_mid(state, alo, ahi, gh, wihT, bih, mW, mb, whhT, bhh):
    return pl.pallas_call(
        _tc_mid_body,
        grid=(GRID,),
        in_specs=[_row_spec(SDIM), _row_spec(HDIM), _row_spec(HDIM),
                  _row_spec(GDIM), _full_spec((SDIM, GDIM)), _full_spec((1, GDIM)),
                  _full_spec((SDIM, SDIM)), _full_spec((1, SDIM)),
                  _full_spec((SDIM, GDIM)), _full_spec((1, GDIM))],
        out_specs=[_row_spec(SDIM), _row_spec(HDIM), _row_spec(HDIM),
                   _row_spec(GDIM)],
        out_shape=[_sds(N_NODES, SDIM), _sds(N_NODES, HDIM),
                   _sds(N_NODES, HDIM), _sds(N_NODES, GDIM)],
    )(state, alo, ahi, gh, wihT, bih, mW, mb, whhT, bhh)


def _tc_final(state, alo, ahi, gh, wihT, bih, muW, mub, lsW, lsb):
    return pl.pallas_call(
        _tc_final_body,
        grid=(GRID,),
        in_specs=[_row_spec(SDIM), _row_spec(HDIM), _row_spec(HDIM),
                  _row_spec(GDIM), _full_spec((SDIM, GDIM)), _full_spec((1, GDIM)),
                  _full_spec((SDIM, LDIM)), _full_spec((1, LDIM)),
                  _full_spec((SDIM, LDIM)), _full_spec((1, LDIM))],
        out_specs=[_row_spec(LDIM), _row_spec(LDIM)],
        out_shape=[_sds(N_NODES, LDIM), _sds(N_NODES, LDIM)],
    )(state, alo, ahi, gh, wihT, bih, muW, mub, lsW, lsb)


def _sc_body(mlo_hbm, mhi_hbm, idx_hbm, zero_hbm, alo_hbm, ahi_hbm,
             acc, iv0, iv1, rows0, rows1, gsem, ssem):
    c = lax.axis_index("c")
    s = lax.axis_index("s")
    stripe = s * ZERO_PER_TILE

    pltpu.sync_copy(zero_hbm.at[pl.ds(stripe, ZERO_PER_TILE)],
                    acc.at[pl.ds(stripe, ZERO_PER_TILE)])
    plsc.subcore_barrier()

    def _accumulate(msg_ref):
        cbase = s * CHUNKS_PER_TILE

        def _fire_block(iv, rows, blk0):
            pltpu.sync_copy(idx_hbm.at[pl.ds(blk0, BLK)], iv)
            for j in range(BLK):
                pltpu.async_copy(msg_ref.at[iv.at[j, 0]], rows.at[j], gsem)

        def _drain_g_fire_s(iv, rows):
            for j in range(BLK):
                pltpu.make_async_copy(msg_ref.at[iv.at[j, 0]], rows.at[j],
                                      gsem).wait()
                pltpu.async_copy(rows.at[j], acc.at[iv.at[j, 1]], ssem,
                                 add=True)

        def _drain_s(iv, rows):
            for j in range(BLK):
                pltpu.make_async_copy(rows.at[j], acc.at[iv.at[j, 1]],
                                      ssem).wait()

        _fire_block(iv0, rows0, cbase)
        _fire_block(iv1, rows1, cbase + BLK)

        def _pair(p, _):
            b0 = cbase + (2 * p) * BLK
            _drain_g_fire_s(iv0, rows0)
            _drain_g_fire_s(iv1, rows1)
            _drain_s(iv0, rows0)
            _fire_block(iv0, rows0, b0 + 2 * BLK)
            _drain_s(iv1, rows1)
            _fire_block(iv1, rows1, b0 + 3 * BLK)
            return 0
        lax.fori_loop(0, BLOCKS_PER_TILE // 2 - 1, _pair, 0)
        _drain_g_fire_s(iv0, rows0)
        _drain_g_fire_s(iv1, rows1)
        _drain_s(iv0, rows0)
        _drain_s(iv1, rows1)

    @pl.when(c == 0)
    def _():
        _accumulate(mlo_hbm)

    @pl.when(c == 1)
    def _():
        _accumulate(mhi_hbm)

    plsc.subcore_barrier()

    @pl.when(c == 0)
    def _():
        pltpu.sync_copy(acc.at[pl.ds(stripe, ZERO_PER_TILE)],
                        alo_hbm.at[pl.ds(stripe, ZERO_PER_TILE)])

    @pl.when(c == 1)
    def _():
        pltpu.sync_copy(acc.at[pl.ds(stripe, ZERO_PER_TILE)],
                        ahi_hbm.at[pl.ds(stripe, ZERO_PER_TILE)])


@functools.cache
def _sc_aggregate_fn():
    return pl.kernel(
        _sc_body,
        out_type=[_sds(ACC_ROWS, HDIM), _sds(ACC_ROWS, HDIM)],
        mesh=plsc.VectorSubcoreMesh(core_axis_name="c", subcore_axis_name="s"),
        scratch_types=[
            pltpu.VMEM_SHARED((ACC_ROWS, HDIM), jnp.float32),
            pltpu.VMEM((BLK, 2, CHUNK), jnp.int32),
            pltpu.VMEM((BLK, 2, CHUNK), jnp.int32),
            pltpu.VMEM((BLK, CHUNK, HDIM), jnp.float32),
            pltpu.VMEM((BLK, CHUNK, HDIM), jnp.float32),
            pltpu.SemaphoreType.DMA,
            pltpu.SemaphoreType.DMA,
        ],
        compiler_params=pltpu.CompilerParams(use_tc_tiling_on_sc=False),
    )


def _sc_aggregate(mlo, mhi, idx_comb, zeros):
    alo, ahi = _sc_aggregate_fn()(mlo, mhi, idx_comb, zeros)
    return alo[:N_NODES], ahi[:N_NODES]


def kernel(x, edge_index, input_W, input_b, msg_W, msg_b, gru_wih, gru_whh,
           gru_bih, gru_bhh, mu_W, mu_b, ls_W, ls_b):
    pad = E_PAD - N_EDGES
    src = jnp.concatenate([edge_index[0], jnp.zeros((pad,), jnp.int32)])
    dst = jnp.concatenate([edge_index[1],
                           jnp.full((pad,), TRASH_ROW, jnp.int32)])
    idx_comb = jnp.stack([src.reshape(CHUNKS_TOTAL, CHUNK),
                          dst.reshape(CHUNKS_TOTAL, CHUNK)], axis=1)
    zeros = jnp.zeros((ACC_ROWS, HDIM), jnp.float32)

    inb = input_b.reshape(1, SDIM)
    mb = msg_b.reshape(ROUNDS, 1, SDIM)
    bih = gru_bih.reshape(ROUNDS, 1, GDIM)
    bhh = gru_bhh.reshape(ROUNDS, 1, GDIM)
    wihT = jnp.transpose(gru_wih, (0, 2, 1))
    whhT = jnp.transpose(gru_whh, (0, 2, 1))
    mub = mu_b.reshape(1, LDIM)
    lsb = ls_b.reshape(1, LDIM)

    state, mlo, mhi, gh = _tc_init(x, input_W, inb, msg_W[0], mb[0],
                                   whhT[0], bhh[0])
    for r in range(ROUNDS):
        alo, ahi = _sc_aggregate(mlo, mhi, idx_comb, zeros)
        if r < ROUNDS - 1:
            state, mlo, mhi, gh = _tc_mid(state, alo, ahi, gh, wihT[r], bih[r],
                                          msg_W[r + 1], mb[r + 1],
                                          whhT[r + 1], bhh[r + 1])
        else:
            mu, ls = _tc_final(state, alo, ahi, gh, wihT[r], bih[r],
                               mu_W, mub, ls_W, lsb)
    return (mu, ls)


# speedup vs baseline: 11.0555x; 1.0603x over previous
import functools

import jax
import jax.numpy as jnp
from jax import lax
from jax.experimental import pallas as pl
from jax.experimental.pallas import tpu as pltpu
from jax.experimental.pallas import tpu_sc as plsc

N_NODES = 100000
N_EDGES = 1600000
FDIM = 128
SDIM = 32
HDIM = 16
GDIM = 96
LDIM = 16
ROUNDS = 3

NC = 2
NS = 16
CHUNK = 512
BLK = 1

CHUNKS_PER_TILE = -(-N_EDGES // (CHUNK * 2 * BLK * NS)) * 2 * BLK
CHUNKS_TOTAL = CHUNKS_PER_TILE * NS
BLOCKS_PER_TILE = CHUNKS_PER_TILE // BLK
E_PAD = CHUNKS_TOTAL * CHUNK

ACC_ROWS = 100096
ZERO_PER_TILE = ACC_ROWS // NS
TRASH_ROW = N_NODES

ROW_T = 2000
GRID = N_NODES // ROW_T


def _dot(a, b):
    return jnp.dot(a, b, preferred_element_type=jnp.float32)


def _tc_init_body(x_ref, inW_ref, inb_ref, mW_ref, mb_ref, whhT_ref, bhh_ref,
                  state_ref, mlo_ref, mhi_ref, gh_ref):
    st = jnp.maximum(_dot(x_ref[...], inW_ref[...]) + inb_ref[...], 0.0)
    state_ref[...] = st
    m = jnp.maximum(_dot(st, mW_ref[...]) + mb_ref[...], 0.0)
    mlo_ref[...] = m[:, :HDIM]
    mhi_ref[...] = m[:, HDIM:]
    gh_ref[...] = _dot(st, whhT_ref[...]) + bhh_ref[...]


def _gru_new_state(state_ref, alo_ref, ahi_ref, gh_ref, wihT_ref, bih_ref):
    w = wihT_ref[...]
    gi = (_dot(alo_ref[...], w[:HDIM, :]) + _dot(ahi_ref[...], w[HDIM:, :])
          + bih_ref[...])
    h = state_ref[...]
    gh = gh_ref[...]
    r = jax.nn.sigmoid(gi[:, :SDIM] + gh[:, :SDIM])
    z = jax.nn.sigmoid(gi[:, SDIM:2 * SDIM] + gh[:, SDIM:2 * SDIM])
    n = jnp.tanh(gi[:, 2 * SDIM:] + r * gh[:, 2 * SDIM:])
    return h + (1.0 - z) * n + z * h


def _tc_mid_body(state_ref, alo_ref, ahi_ref, gh_ref, wihT_ref, bih_ref,
                 mW_ref, mb_ref, whhT_ref, bhh_ref,
                 nstate_ref, mlo_ref, mhi_ref, ghn_ref):
    new = _gru_new_state(state_ref, alo_ref, ahi_ref, gh_ref, wihT_ref, bih_ref)
    nstate_ref[...] = new
    m = jnp.maximum(_dot(new, mW_ref[...]) + mb_ref[...], 0.0)
    mlo_ref[...] = m[:, :HDIM]
    mhi_ref[...] = m[:, HDIM:]
    ghn_ref[...] = _dot(new, whhT_ref[...]) + bhh_ref[...]


def _tc_final_body(state_ref, alo_ref, ahi_ref, gh_ref, wihT_ref, bih_ref,
                   muW_ref, mub_ref, lsW_ref, lsb_ref, mu_ref, ls_ref):
    new = _gru_new_state(state_ref, alo_ref, ahi_ref, gh_ref, wihT_ref, bih_ref)
    mu_ref[...] = _dot(new, muW_ref[...]) + mub_ref[...]
    ls_ref[...] = _dot(new, lsW_ref[...]) + lsb_ref[...]


def _row_spec(width):
    return pl.BlockSpec((ROW_T, width), lambda i: (i, 0))


def _full_spec(shape):
    return pl.BlockSpec(shape, lambda i: (0,) * len(shape))


def _sds(*shape):
    return jax.ShapeDtypeStruct(shape, jnp.float32)


def _tc_init(x, inW, inb, mW, mb, whhT, bhh):
    return pl.pallas_call(
        _tc_init_body,
        grid=(GRID,),
        in_specs=[_row_spec(FDIM), _full_spec((FDIM, SDIM)), _full_spec((1, SDIM)),
                  _full_spec((SDIM, SDIM)), _full_spec((1, SDIM)),
                  _full_spec((SDIM, GDIM)), _full_spec((1, GDIM))],
        out_specs=[_row_spec(SDIM), _row_spec(HDIM), _row_spec(HDIM),
                   _row_spec(GDIM)],
        out_shape=[_sds(N_NODES, SDIM), _sds(N_NODES, HDIM),
                   _sds(N_NODES, HDIM), _sds(N_NODES, GDIM)],
    )(x, inW, inb, mW, mb, whhT, bhh)


def _tc_mid(state, alo, ahi, gh, wihT, bih, mW, mb, whhT, bhh):
    return pl.pallas_call(
        _tc_mid_body,
        grid=(GRID,),
        in_specs=[_row_spec(SDIM), _row_spec(HDIM), _row_spec(HDIM),
                  _row_spec(GDIM), _full_spec((SDIM, GDIM)), _full_spec((1, GDIM)),
                  _full_spec((SDIM, SDIM)), _full_spec((1, SDIM)),
                  _full_spec((SDIM, GDIM)), _full_spec((1, GDIM))],
        out_specs=[_row_spec(SDIM), _row_spec(HDIM), _row_spec(HDIM),
                   _row_spec(GDIM)],
        out_shape=[_sds(N_NODES, SDIM), _sds(N_NODES, HDIM),
                   _sds(N_NODES, HDIM), _sds(N_NODES, GDIM)],
    )(state, alo, ahi, gh, wihT, bih, mW, mb, whhT, bhh)


def _tc_final(state, alo, ahi, gh, wihT, bih, muW, mub, lsW, lsb):
    return pl.pallas_call(
        _tc_final_body,
        grid=(GRID,),
        in_specs=[_row_spec(SDIM), _row_spec(HDIM), _row_spec(HDIM),
                  _row_spec(GDIM), _full_spec((SDIM, GDIM)), _full_spec((1, GDIM)),
                  _full_spec((SDIM, LDIM)), _full_spec((1, LDIM)),
                  _full_spec((SDIM, LDIM)), _full_spec((1, LDIM))],
        out_specs=[_row_spec(LDIM), _row_spec(LDIM)],
        out_shape=[_sds(N_NODES, LDIM), _sds(N_NODES, LDIM)],
    )(state, alo, ahi, gh, wihT, bih, muW, mub, lsW, lsb)


def _sc_body(mlo_hbm, mhi_hbm, idx_hbm, zero_hbm, alo_hbm, ahi_hbm,
             acc, iv0, iv1, rows0, rows1, gsem, ssem):
    c = lax.axis_index("c")
    s = lax.axis_index("s")
    stripe = s * ZERO_PER_TILE

    pltpu.sync_copy(zero_hbm.at[pl.ds(stripe, ZERO_PER_TILE)],
                    acc.at[pl.ds(stripe, ZERO_PER_TILE)])
    plsc.subcore_barrier()

    def _accumulate(msg_ref):
        cbase = s * CHUNKS_PER_TILE

        def _fire_block(iv, rows, blk0):
            pltpu.sync_copy(idx_hbm.at[pl.ds(blk0, BLK)], iv)
            for j in range(BLK):
                pltpu.async_copy(msg_ref.at[iv.at[j, 0]], rows.at[j], gsem)

        def _drain_g_fire_s(iv, rows):
            for j in range(BLK):
                pltpu.make_async_copy(msg_ref.at[iv.at[j, 0]], rows.at[j],
                                      gsem).wait()
                pltpu.async_copy(rows.at[j], acc.at[iv.at[j, 1]], ssem,
                                 add=True)

        def _drain_s(iv, rows):
            for j in range(BLK):
                pltpu.make_async_copy(rows.at[j], acc.at[iv.at[j, 1]],
                                      ssem).wait()

        _fire_block(iv0, rows0, cbase)
        _fire_block(iv1, rows1, cbase + BLK)

        def _pair(p, _):
            b0 = cbase + (2 * p) * BLK
            _drain_g_fire_s(iv0, rows0)
            _drain_g_fire_s(iv1, rows1)
            _drain_s(iv0, rows0)
            _fire_block(iv0, rows0, b0 + 2 * BLK)
            _drain_s(iv1, rows1)
            _fire_block(iv1, rows1, b0 + 3 * BLK)
            return 0
        lax.fori_loop(0, BLOCKS_PER_TILE // 2 - 1, _pair, 0)
        _drain_g_fire_s(iv0, rows0)
        _drain_g_fire_s(iv1, rows1)
        _drain_s(iv0, rows0)
        _drain_s(iv1, rows1)

    @pl.when(c == 0)
    def _():
        _accumulate(mlo_hbm)

    @pl.when(c == 1)
    def _():
        _accumulate(mhi_hbm)

    plsc.subcore_barrier()

    @pl.when(c == 0)
    def _():
        pltpu.sync_copy(acc.at[pl.ds(stripe, ZERO_PER_TILE)],
                        alo_hbm.at[pl.ds(stripe, ZERO_PER_TILE)])

    @pl.when(c == 1)
    def _():
        pltpu.sync_copy(acc.at[pl.ds(stripe, ZERO_PER_TILE)],
                        ahi_hbm.at[pl.ds(stripe, ZERO_PER_TILE)])


@functools.cache
def _sc_aggregate_fn():
    return pl.kernel(
        _sc_body,
        out_type=[_sds(ACC_ROWS, HDIM), _sds(ACC_ROWS, HDIM)],
        mesh=plsc.VectorSubcoreMesh(core_axis_name="c", subcore_axis_name="s"),
        scratch_types=[
            pltpu.VMEM_SHARED((ACC_ROWS, HDIM), jnp.float32),
            pltpu.VMEM((BLK, 2, CHUNK), jnp.int32),
            pltpu.VMEM((BLK, 2, CHUNK), jnp.int32),
            pltpu.VMEM((BLK, CHUNK, HDIM), jnp.float32),
            pltpu.VMEM((BLK, CHUNK, HDIM), jnp.float32),
            pltpu.SemaphoreType.DMA,
            pltpu.SemaphoreType.DMA,
        ],
        compiler_params=pltpu.CompilerParams(use_tc_tiling_on_sc=False),
    )


def _sc_aggregate(mlo, mhi, idx_comb, zeros):
    alo, ahi = _sc_aggregate_fn()(mlo, mhi, idx_comb, zeros)
    return alo[:N_NODES], ahi[:N_NODES]


def kernel(x, edge_index, input_W, input_b, msg_W, msg_b, gru_wih, gru_whh,
           gru_bih, gru_bhh, mu_W, mu_b, ls_W, ls_b):
    pad = E_PAD - N_EDGES
    src = jnp.concatenate([edge_index[0], jnp.zeros((pad,), jnp.int32)])
    dst = jnp.concatenate([edge_index[1],
                           jnp.full((pad,), TRASH_ROW, jnp.int32)])
    idx_comb = jnp.stack([src.reshape(CHUNKS_TOTAL, CHUNK),
                          dst.reshape(CHUNKS_TOTAL, CHUNK)], axis=1)
    zeros = jnp.zeros((ACC_ROWS, HDIM), jnp.float32)

    inb = input_b.reshape(1, SDIM)
    mb = msg_b.reshape(ROUNDS, 1, SDIM)
    bih = gru_bih.reshape(ROUNDS, 1, GDIM)
    bhh = gru_bhh.reshape(ROUNDS, 1, GDIM)
    wihT = jnp.transpose(gru_wih, (0, 2, 1))
    whhT = jnp.transpose(gru_whh, (0, 2, 1))
    mub = mu_b.reshape(1, LDIM)
    lsb = ls_b.reshape(1, LDIM)

    state, mlo, mhi, gh = _tc_init(x, input_W, inb, msg_W[0], mb[0],
                                   whhT[0], bhh[0])
    for r in range(ROUNDS):
        alo, ahi = _sc_aggregate(mlo, mhi, idx_comb, zeros)
        if r < ROUNDS - 1:
            state, mlo, mhi, gh = _tc_mid(state, alo, ahi, gh, wihT[r], bih[r],
                                          msg_W[r + 1], mb[r + 1],
                                          whhT[r + 1], bhh[r + 1])
        else:
            mu, ls = _tc_final(state, alo, ahi, gh, wihT[r], bih[r],
                               mu_W, mub, ls_W, lsb)
    return (mu, ls)
